# Initial kernel scaffold; baseline (speedup 1.0000x reference)
import jax
import jax.numpy as jnp
from jax.experimental import pallas as pl


def kernel(input, targets):
    raise NotImplementedError("write your pallas kernel here")


# speedup vs baseline: 3.0136x; 3.0136x over previous
import functools

import jax
import jax.numpy as jnp
import numpy as np
from jax import lax
from jax.experimental import pallas as pl
from jax.experimental.pallas import tpu as pltpu
from jax.experimental.pallas import tpu_sc as plsc

BS = 16
NA = 3
NCLS = 80
BB = 5 + NCLS
HW = 52
S = HW * HW
R = NA * BB * S
N1 = float(BS * NA * S)
T = 50
TP = 64
NROW = 44

_AW = (np.float32(1.25), np.float32(2.0), np.float32(4.125))
_AH = (np.float32(1.625), np.float32(3.75), np.float32(2.875))
_AA = tuple(np.float32(a * b) for a, b in zip(_AW, _AH))
_EPS16 = np.float32(1e-16)
_EPS12 = np.float32(1e-12)


def _sc_assign_body(flat_in, tgt_hbm, gath_out, meta_out,
                    tgt_v, idx_v, out_v, met_v, sem):
    cidx = lax.axis_index("c")
    b = lax.axis_index("s")

    @pl.when(cidx == 0)
    def _():
        pltpu.sync_copy(tgt_hbm.at[b], tgt_v)
        bR = b * R

        one_i = jnp.ones((16,), jnp.int32)
        zero_i = jnp.zeros((16,), jnp.int32)
        Kc, Vc, Z0c, Z1c, Sc, Cc, ABc, PAc = [], [], [], [], [], [], [], []
        for ch in range(4):
            sl = pl.ds(ch * 16, 16)
            t0 = tgt_v[0, sl]
            t1 = tgt_v[1, sl]
            t2 = tgt_v[2, sl]
            t3 = tgt_v[3, sl]
            t4 = tgt_v[4, sl]
            valid = jnp.where((t0 + t1 + t2 + t3 + t4) != 0.0, one_i, zero_i)
            gx = t1 * np.float32(HW)
            gy = t2 * np.float32(HW)
            gw = t3 * np.float32(HW)
            gh = t4 * np.float32(HW)
            gi = gx.astype(jnp.int32)
            gj = gy.astype(jnp.int32)
            tx = gx - gi.astype(jnp.float32)
            ty = gy - gj.astype(jnp.float32)
            ious = []
            for k in range(3):
                inter = jnp.maximum(
                    jnp.minimum(gw, _AW[k]) * jnp.minimum(gh, _AH[k]),
                    np.float32(0.0))
                union = gw * gh + _AA[k] - inter
                ious.append(inter / (union + _EPS16))
            best01 = jnp.where(ious[1] > ious[0], one_i, zero_i)
            m01 = jnp.where(ious[1] > ious[0], ious[1], ious[0])
            best = jnp.where(ious[2] > m01, 2 * one_i, best01)
            awb = jnp.where(best == 0, _AW[0], jnp.where(best == 1, _AW[1], _AW[2]))
            ahb = jnp.where(best == 0, _AH[0], jnp.where(best == 1, _AH[1], _AH[2]))
            rw = gw / awb + _EPS16
            rh = gh / ahb + _EPS16
            ssp = gj * HW + gi
            K = best * S + ssp
            cls = t0.astype(jnp.int32)
            n0 = jnp.where(ious[0] > np.float32(0.5), one_i, zero_i)
            n1 = jnp.where(ious[1] > np.float32(0.5), one_i, zero_i)
            n2 = jnp.where(ious[2] > np.float32(0.5), one_i, zero_i)
            nsum = n0 + n1 + n2
            z1 = jnp.where(nsum > 0, one_i, zero_i)
            z0 = jnp.where(nsum < 3, one_i, zero_i)
            met_v[pl.ds(64 + ch * 16, 16)] = tx
            met_v[pl.ds(128 + ch * 16, 16)] = ty
            met_v[pl.ds(192 + ch * 16, 16)] = rw
            met_v[pl.ds(256 + ch * 16, 16)] = rh
            Kc.append(K)
            Vc.append(valid)
            Z0c.append(z0)
            Z1c.append(z1)
            Sc.append(ssp)
            Cc.append(cls)
            ABc.append(bR + best * (BB * S) + ssp)
            PAc.append(bR + (best * BB + 5 + cls) * S + ssp)
        iotas = [lax.iota(jnp.int32, 16) + ch * 16 for ch in range(4)]

        PKr = [Kc[ch] * 128 + Cc[ch] for ch in range(4)]

        def make_killer(sch):
            def killer(tq, carry):
                kk = list(carry[0:4])
                kp = list(carry[4:8])
                k0 = list(carry[8:12])
                k1 = list(carry[12:16])
                tqv = jnp.full((16,), tq, jnp.int32)
                lane = tqv - sch * 16

                def pick(reg):
                    return reg.at[lane].get(mode="promise_in_bounds")

                kq = pick(Kc[sch])
                pq = pick(PKr[sch])
                sq = pick(Sc[sch])
                vq = pick(Vc[sch])
                zq0 = pick(Z0c[sch]) * vq
                zq1 = pick(Z1c[sch]) * vq
                for ch in range(4):
                    earlier = jnp.where(iotas[ch] < tqv, one_i, zero_i)
                    eqk = jnp.where(Kc[ch] == kq, one_i, zero_i) * earlier
                    eqp = jnp.where(PKr[ch] == pq, one_i, zero_i) * earlier
                    eqs = jnp.where(Sc[ch] == sq, one_i, zero_i) * earlier
                    kk[ch] = kk[ch] | (eqk * vq)
                    kp[ch] = kp[ch] | (eqp * vq)
                    k0[ch] = k0[ch] | (eqs * zq0 * Z0c[ch])
                    k1[ch] = k1[ch] | (eqs * zq1 * Z1c[ch])
                return tuple(kk + kp + k0 + k1)
            return killer

        carry = tuple([zero_i] * 16)
        for sch in range(4):
            lo = max(sch * 16, 1)
            hi = min(sch * 16 + 16, T)
            carry = lax.fori_loop(lo, hi, make_killer(sch), carry)

        for ch in range(4):
            sl = pl.ds(ch * 16, 16)
            win = Vc[ch] * (1 - carry[ch])
            pair = Vc[ch] * (1 - carry[4 + ch])
            nz0 = Vc[ch] * Z0c[ch] * (1 - carry[8 + ch])
            nz1 = Vc[ch] * Z1c[ch] * (1 - carry[12 + ch])
            met_v[sl] = win.astype(jnp.float32)
            met_v[pl.ds(320 + ch * 16, 16)] = pair.astype(jnp.float32)
            met_v[pl.ds(384 + ch * 16, 16)] = nz0.astype(jnp.float32)
            met_v[pl.ds(448 + ch * 16, 16)] = nz1.astype(jnp.float32)

        for c in range(BB):
            for ch in range(4):
                p = c * 64 + ch * 16
                idx_v[p // 128, pl.ds(p % 128, 16)] = ABc[ch] + c * S
        for ch in range(4):
            idx_v[42, pl.ds(64 + ch * 16, 16)] = PAc[ch]
            idx_v[43, pl.ds(ch * 16, 16)] = bR + 4 * S + Sc[ch]
            idx_v[43, pl.ds(64 + ch * 16, 16)] = bR + (BB + 4) * S + Sc[ch]

        cps = [pltpu.async_copy(flat_in.at[idx_v.at[r]], out_v.at[r], sem)
               for r in range(NROW)]
        for cp in cps:
            cp.wait()
        pltpu.sync_copy(out_v, gath_out.at[b])
        pltpu.sync_copy(met_v, meta_out.at[b])


_sc_assign = pl.kernel(
    _sc_assign_body,
    out_type=(
        jax.ShapeDtypeStruct((BS, NROW, 128), jnp.float32),
        jax.ShapeDtypeStruct((BS, 512), jnp.float32),
    ),
    mesh=plsc.VectorSubcoreMesh(
        core_axis_name="c", subcore_axis_name="s", num_cores=2, num_subcores=16),
    scratch_types=[
        pltpu.VMEM((5, TP), jnp.float32),
        pltpu.VMEM((NROW, 128), jnp.int32),
        pltpu.VMEM((NROW, 128), jnp.float32),
        pltpu.VMEM((512,), jnp.float32),
        pltpu.SemaphoreType.DMA,
    ],
)


def _sigm(x):
    return jnp.where(x >= 0,
                     1.0 / (1.0 + jnp.exp(-x)),
                     jnp.exp(x) / (1.0 + jnp.exp(x)))


def _tc_loss_body(conf_ref, gath_ref, meta_ref, out_ref):
    one = np.float32(1.0)
    confl = conf_ref[...]
    pall = jnp.clip(_sigm(confl), _EPS12, one)
    dense = -jnp.sum(jnp.log(one - pall))

    g = gath_ref[...]
    m = meta_ref[...]
    win = m[:, 0]
    tx = m[:, 1]
    ty = m[:, 2]
    rw = jnp.where(win > 0, m[:, 3], one)
    rh = jnp.where(win > 0, m[:, 4], one)
    pairf = m[:, 5]
    nz0f = m[:, 6]
    nz1f = m[:, 7]

    def safe(v, flag):
        return jnp.where(flag > 0, v, np.float32(0.0))

    xl = safe(g[:, 0], win)
    yl = safe(g[:, 1], win)
    wl = g[:, 2]
    hl = g[:, 3]
    cfl = safe(g[:, 4], win)
    clsl = jnp.where(win[:, None, :] > 0, g[:, 5:85], np.float32(0.0))
    pairv = safe(g[:, 85], pairf)
    nz0v = safe(g[:, 86], nz0f)
    nz1v = safe(g[:, 87], nz1f)

    px = jnp.clip(_sigm(xl), _EPS12, one)
    py = jnp.clip(_sigm(yl), _EPS12, one)
    sx = jnp.sum(win * -(tx * jnp.log(px) + (one - tx) * jnp.log(one - px)))
    sy = jnp.sum(win * -(ty * jnp.log(py) + (one - ty) * jnp.log(one - py)))
    sw = jnp.sum(win * (wl - jnp.log(rw)) ** 2)
    sh = jnp.sum(win * (hl - jnp.log(rh)) ** 2)
    pcf = jnp.clip(_sigm(cfl), _EPS12, one)
    sconf = jnp.sum(win * -jnp.log(pcf))
    pcls = jnp.clip(_sigm(clsl), _EPS12, one)
    scls = -jnp.sum(win[:, None, :] * jnp.log(one - pcls))
    ppr = jnp.clip(_sigm(pairv), _EPS12, one)
    scls = scls + jnp.sum(pairf * (-jnp.log(ppr) + jnp.log(one - ppr)))
    p0 = jnp.clip(_sigm(nz0v), _EPS12, one)
    p1 = jnp.clip(_sigm(nz1v), _EPS12, one)
    nzcorr = jnp.sum(nz0f * -jnp.log(one - p0)) + jnp.sum(nz1f * -jnp.log(one - p1))
    nm = jnp.sum(win)

    n1 = np.float32(N1)
    loss_x = sx / n1
    loss_y = sy / n1
    loss_w = sw / n1
    loss_h = sh / n1
    loss_conf = sconf / n1
    loss_nconf = np.float32(0.5) * (dense - nzcorr) / n1
    loss_cls = scls / jnp.maximum(nm * NCLS, one)
    loss = (np.float32(2.5) * (loss_x + loss_y + loss_w + loss_h)
            + np.float32(10.0) * loss_conf + np.float32(3.0) * loss_nconf
            + np.float32(20.0) * loss_cls)

    ii = lax.broadcasted_iota(jnp.int32, (1, 128), 1)
    vals = (loss, loss_x, loss_y, loss_w, loss_h, loss_conf, loss_cls)
    v = jnp.zeros((1, 128), jnp.float32)
    for i, s in enumerate(vals):
        v = v + jnp.where(ii == i, s, np.float32(0.0))
    out_ref[...] = v


def kernel(input, targets):
    conf = input.reshape(BS, NA, BB, S)[:, :, 4, :].reshape(BS * NA, S)
    flat = input.reshape(BS * R)
    tgt = jnp.transpose(targets, (0, 2, 1))
    tgt = jnp.pad(tgt, ((0, 0), (0, 0), (0, TP - T)))
    gath, meta = _sc_assign(flat, tgt)
    gath = gath.reshape(BS, 2 * NROW, 64)
    meta = meta.reshape(BS, 8, 64)
    out = pl.pallas_call(
        _tc_loss_body,
        grid=(),
        in_specs=[
            pl.BlockSpec((BS * NA, S), lambda: (0, 0)),
            pl.BlockSpec((BS, 2 * NROW, 64), lambda: (0, 0, 0)),
            pl.BlockSpec((BS, 8, 64), lambda: (0, 0, 0)),
        ],
        out_specs=pl.BlockSpec((1, 128), lambda: (0, 0)),
        out_shape=jax.ShapeDtypeStruct((1, 128), jnp.float32),
    )(conf, gath, meta)
    o = out[0]
    return (o[0], o[1], o[2], o[3], o[4], o[5], o[6])


# speedup vs baseline: 3.9575x; 1.3132x over previous
import functools

import jax
import jax.numpy as jnp
import numpy as np
from jax import lax
from jax.experimental import pallas as pl
from jax.experimental.pallas import tpu as pltpu
from jax.experimental.pallas import tpu_sc as plsc

BS = 16
NA = 3
NCLS = 80
BB = 5 + NCLS
HW = 52
S = HW * HW
R = NA * BB * S
N1 = float(BS * NA * S)
T = 50
TP = 64
NROW = 44

_AW = (np.float32(1.25), np.float32(2.0), np.float32(4.125))
_AH = (np.float32(1.625), np.float32(3.75), np.float32(2.875))
_AA = tuple(np.float32(a * b) for a, b in zip(_AW, _AH))
_EPS16 = np.float32(1e-16)
_EPS12 = np.float32(1e-12)


def _sc_assign_body(flat_in, tgt_hbm, gath_out, meta_out,
                    tgt_v, idx_v, out_v, met_v, sem):
    cidx = lax.axis_index("c")
    b = lax.axis_index("s")

    @pl.when(cidx == 0)
    def _():
        pltpu.sync_copy(tgt_hbm.at[b], tgt_v)
        bR = b * R

        one_i = jnp.ones((16,), jnp.int32)
        zero_i = jnp.zeros((16,), jnp.int32)
        Kc, Vc, Z0c, Z1c, Sc, Cc, ABc, PAc = [], [], [], [], [], [], [], []
        for ch in range(4):
            sl = pl.ds(ch * 16, 16)
            t0 = tgt_v[0, sl]
            t1 = tgt_v[1, sl]
            t2 = tgt_v[2, sl]
            t3 = tgt_v[3, sl]
            t4 = tgt_v[4, sl]
            valid = jnp.where((t0 + t1 + t2 + t3 + t4) != 0.0, one_i, zero_i)
            gx = t1 * np.float32(HW)
            gy = t2 * np.float32(HW)
            gw = t3 * np.float32(HW)
            gh = t4 * np.float32(HW)
            gi = gx.astype(jnp.int32)
            gj = gy.astype(jnp.int32)
            tx = gx - gi.astype(jnp.float32)
            ty = gy - gj.astype(jnp.float32)
            ious = []
            for k in range(3):
                inter = jnp.maximum(
                    jnp.minimum(gw, _AW[k]) * jnp.minimum(gh, _AH[k]),
                    np.float32(0.0))
                union = gw * gh + _AA[k] - inter
                ious.append(inter / (union + _EPS16))
            best01 = jnp.where(ious[1] > ious[0], one_i, zero_i)
            m01 = jnp.where(ious[1] > ious[0], ious[1], ious[0])
            best = jnp.where(ious[2] > m01, 2 * one_i, best01)
            awb = jnp.where(best == 0, _AW[0], jnp.where(best == 1, _AW[1], _AW[2]))
            ahb = jnp.where(best == 0, _AH[0], jnp.where(best == 1, _AH[1], _AH[2]))
            rw = gw / awb + _EPS16
            rh = gh / ahb + _EPS16
            ssp = gj * HW + gi
            K = best * S + ssp
            cls = t0.astype(jnp.int32)
            n0 = jnp.where(ious[0] > np.float32(0.5), one_i, zero_i)
            n1 = jnp.where(ious[1] > np.float32(0.5), one_i, zero_i)
            n2 = jnp.where(ious[2] > np.float32(0.5), one_i, zero_i)
            nsum = n0 + n1 + n2
            z1 = jnp.where(nsum > 0, one_i, zero_i)
            z0 = jnp.where(nsum < 3, one_i, zero_i)
            met_v[pl.ds(64 + ch * 16, 16)] = tx
            met_v[pl.ds(128 + ch * 16, 16)] = ty
            met_v[pl.ds(192 + ch * 16, 16)] = rw
            met_v[pl.ds(256 + ch * 16, 16)] = rh
            Kc.append(K)
            Vc.append(valid)
            Z0c.append(z0)
            Z1c.append(z1)
            Sc.append(ssp)
            Cc.append(cls)
            ABc.append(bR + best * (BB * S) + ssp)
            PAc.append(bR + (best * BB + 5 + cls) * S + ssp)
        iotas = [lax.iota(jnp.int32, 16) + ch * 16 for ch in range(4)]

        PKr = [Kc[ch] * 128 + Cc[ch] for ch in range(4)]

        def make_killer(sch):
            def killer(tq, carry):
                kk = list(carry[0:4])
                kp = list(carry[4:8])
                k0 = list(carry[8:12])
                k1 = list(carry[12:16])
                tqv = jnp.full((16,), tq, jnp.int32)
                lane = tqv - sch * 16

                def pick(reg):
                    return reg.at[lane].get(mode="promise_in_bounds")

                kq = pick(Kc[sch])
                pq = pick(PKr[sch])
                sq = pick(Sc[sch])
                vq = pick(Vc[sch])
                zq0 = pick(Z0c[sch]) * vq
                zq1 = pick(Z1c[sch]) * vq
                for ch in range(4):
                    earlier = jnp.where(iotas[ch] < tqv, one_i, zero_i)
                    eqk = jnp.where(Kc[ch] == kq, one_i, zero_i) * earlier
                    eqp = jnp.where(PKr[ch] == pq, one_i, zero_i) * earlier
                    eqs = jnp.where(Sc[ch] == sq, one_i, zero_i) * earlier
                    kk[ch] = kk[ch] | (eqk * vq)
                    kp[ch] = kp[ch] | (eqp * vq)
                    k0[ch] = k0[ch] | (eqs * zq0 * Z0c[ch])
                    k1[ch] = k1[ch] | (eqs * zq1 * Z1c[ch])
                return tuple(kk + kp + k0 + k1)
            return killer

        carry = tuple([zero_i] * 16)
        for sch in range(4):
            lo = max(sch * 16, 1)
            hi = min(sch * 16 + 16, T)
            carry = lax.fori_loop(lo, hi, make_killer(sch), carry)

        for ch in range(4):
            sl = pl.ds(ch * 16, 16)
            win = Vc[ch] * (1 - carry[ch])
            pair = Vc[ch] * (1 - carry[4 + ch])
            nz0 = Vc[ch] * Z0c[ch] * (1 - carry[8 + ch])
            nz1 = Vc[ch] * Z1c[ch] * (1 - carry[12 + ch])
            met_v[sl] = win.astype(jnp.float32)
            met_v[pl.ds(320 + ch * 16, 16)] = pair.astype(jnp.float32)
            met_v[pl.ds(384 + ch * 16, 16)] = nz0.astype(jnp.float32)
            met_v[pl.ds(448 + ch * 16, 16)] = nz1.astype(jnp.float32)

        for c in range(BB):
            for ch in range(4):
                p = c * 64 + ch * 16
                idx_v[p // 128, pl.ds(p % 128, 16)] = ABc[ch] + c * S
        for ch in range(4):
            idx_v[42, pl.ds(64 + ch * 16, 16)] = PAc[ch]
            idx_v[43, pl.ds(ch * 16, 16)] = bR + 4 * S + Sc[ch]
            idx_v[43, pl.ds(64 + ch * 16, 16)] = bR + (BB + 4) * S + Sc[ch]

        cps = [pltpu.async_copy(flat_in.at[idx_v.at[r]], out_v.at[r], sem)
               for r in range(NROW)]
        for cp in cps:
            cp.wait()
        pltpu.sync_copy(out_v, gath_out.at[b])
        pltpu.sync_copy(met_v, meta_out.at[b])


_sc_assign = pl.kernel(
    _sc_assign_body,
    out_type=(
        jax.ShapeDtypeStruct((BS, NROW, 128), jnp.float32),
        jax.ShapeDtypeStruct((BS, 512), jnp.float32),
    ),
    mesh=plsc.VectorSubcoreMesh(
        core_axis_name="c", subcore_axis_name="s", num_cores=2, num_subcores=16),
    scratch_types=[
        pltpu.VMEM((5, TP), jnp.float32),
        pltpu.VMEM((NROW, 128), jnp.int32),
        pltpu.VMEM((NROW, 128), jnp.float32),
        pltpu.VMEM((512,), jnp.float32),
        pltpu.SemaphoreType.DMA,
    ],
)


def _sigm(x):
    return jnp.where(x >= 0,
                     1.0 / (1.0 + jnp.exp(-x)),
                     jnp.exp(x) / (1.0 + jnp.exp(x)))


def _tc_loss_body(inp_ref, gath_ref, meta_ref, out_ref, conf_v, sem):
    cps = [pltpu.make_async_copy(
        inp_ref.at[i * BB + 4], conf_v.at[i], sem) for i in range(BS * NA)]
    for cp in cps:
        cp.start()
    for cp in cps:
        cp.wait()
    one = np.float32(1.0)
    confl = conf_v[...]
    pall = jnp.clip(_sigm(confl), _EPS12, one)
    dense = -jnp.sum(jnp.log(one - pall))

    g = gath_ref[...]
    m = meta_ref[...]
    win = m[:, 0]
    tx = m[:, 1]
    ty = m[:, 2]
    rw = jnp.where(win > 0, m[:, 3], one)
    rh = jnp.where(win > 0, m[:, 4], one)
    pairf = m[:, 5]
    nz0f = m[:, 6]
    nz1f = m[:, 7]

    def safe(v, flag):
        return jnp.where(flag > 0, v, np.float32(0.0))

    xl = safe(g[:, 0], win)
    yl = safe(g[:, 1], win)
    wl = g[:, 2]
    hl = g[:, 3]
    cfl = safe(g[:, 4], win)
    clsl = jnp.where(win[:, None, :] > 0, g[:, 5:85], np.float32(0.0))
    pairv = safe(g[:, 85], pairf)
    nz0v = safe(g[:, 86], nz0f)
    nz1v = safe(g[:, 87], nz1f)

    px = jnp.clip(_sigm(xl), _EPS12, one)
    py = jnp.clip(_sigm(yl), _EPS12, one)
    sx = jnp.sum(win * -(tx * jnp.log(px) + (one - tx) * jnp.log(one - px)))
    sy = jnp.sum(win * -(ty * jnp.log(py) + (one - ty) * jnp.log(one - py)))
    sw = jnp.sum(win * (wl - jnp.log(rw)) ** 2)
    sh = jnp.sum(win * (hl - jnp.log(rh)) ** 2)
    pcf = jnp.clip(_sigm(cfl), _EPS12, one)
    sconf = jnp.sum(win * -jnp.log(pcf))
    pcls = jnp.clip(_sigm(clsl), _EPS12, one)
    scls = -jnp.sum(win[:, None, :] * jnp.log(one - pcls))
    ppr = jnp.clip(_sigm(pairv), _EPS12, one)
    scls = scls + jnp.sum(pairf * (-jnp.log(ppr) + jnp.log(one - ppr)))
    p0 = jnp.clip(_sigm(nz0v), _EPS12, one)
    p1 = jnp.clip(_sigm(nz1v), _EPS12, one)
    nzcorr = jnp.sum(nz0f * -jnp.log(one - p0)) + jnp.sum(nz1f * -jnp.log(one - p1))
    nm = jnp.sum(win)

    n1 = np.float32(N1)
    loss_x = sx / n1
    loss_y = sy / n1
    loss_w = sw / n1
    loss_h = sh / n1
    loss_conf = sconf / n1
    loss_nconf = np.float32(0.5) * (dense - nzcorr) / n1
    loss_cls = scls / jnp.maximum(nm * NCLS, one)
    loss = (np.float32(2.5) * (loss_x + loss_y + loss_w + loss_h)
            + np.float32(10.0) * loss_conf + np.float32(3.0) * loss_nconf
            + np.float32(20.0) * loss_cls)

    ii = lax.broadcasted_iota(jnp.int32, (1, 128), 1)
    vals = (loss, loss_x, loss_y, loss_w, loss_h, loss_conf, loss_cls)
    v = jnp.zeros((1, 128), jnp.float32)
    for i, s in enumerate(vals):
        v = v + jnp.where(ii == i, s, np.float32(0.0))
    out_ref[...] = v


def kernel(input, targets):
    rows = input.reshape(BS * NA * BB, S)
    flat = input.reshape(BS * R)
    tgt = jnp.transpose(targets, (0, 2, 1))
    tgt = jnp.pad(tgt, ((0, 0), (0, 0), (0, TP - T)))
    gath, meta = _sc_assign(flat, tgt)
    gath = gath.reshape(BS, 2 * NROW, 64)
    meta = meta.reshape(BS, 8, 64)
    out = pl.pallas_call(
        _tc_loss_body,
        grid=(),
        in_specs=[
            pl.BlockSpec(memory_space=pl.ANY),
            pl.BlockSpec((BS, 2 * NROW, 64), lambda: (0, 0, 0)),
            pl.BlockSpec((BS, 8, 64), lambda: (0, 0, 0)),
        ],
        out_specs=pl.BlockSpec((1, 128), lambda: (0, 0)),
        out_shape=jax.ShapeDtypeStruct((1, 128), jnp.float32),
        scratch_shapes=[
            pltpu.VMEM((BS * NA, S), jnp.float32),
            pltpu.SemaphoreType.DMA,
        ],
    )(rows, gath, meta)
    o = out[0]
    return (o[0], o[1], o[2], o[3], o[4], o[5], o[6])


# speedup vs baseline: 5.0126x; 1.2666x over previous
import functools

import jax
import jax.numpy as jnp
import numpy as np
from jax import lax
from jax.experimental import pallas as pl
from jax.experimental.pallas import tpu as pltpu
from jax.experimental.pallas import tpu_sc as plsc

BS = 16
NA = 3
NCLS = 80
BB = 5 + NCLS
HW = 52
S = HW * HW
R = NA * BB * S
N1 = float(BS * NA * S)
T = 50
TP = 64
NROW = 44
CPAD = S + 112

_AW = (np.float32(1.25), np.float32(2.0), np.float32(4.125))
_AH = (np.float32(1.625), np.float32(3.75), np.float32(2.875))
_AA = tuple(np.float32(a * b) for a, b in zip(_AW, _AH))
_EPS16 = np.float32(1e-16)
_EPS12 = np.float32(1e-12)


def _sc_assign_body(flat_in, tgt_hbm, gath_out, meta_out,
                    tgt_v, idx_v, out_v, met_v, sem):
    cidx = lax.axis_index("c")
    b = lax.axis_index("s")

    @pl.when(cidx == 0)
    def _():
        pltpu.sync_copy(tgt_hbm.at[b], tgt_v)
        bR = b * R

        one_i = jnp.ones((16,), jnp.int32)
        zero_i = jnp.zeros((16,), jnp.int32)
        Kc, Vc, Z0c, Z1c, Sc, Cc, ABc, PAc = [], [], [], [], [], [], [], []
        for ch in range(4):
            sl = pl.ds(ch * 16, 16)
            t0 = tgt_v[0, sl]
            t1 = tgt_v[1, sl]
            t2 = tgt_v[2, sl]
            t3 = tgt_v[3, sl]
            t4 = tgt_v[4, sl]
            valid = jnp.where((t0 + t1 + t2 + t3 + t4) != 0.0, one_i, zero_i)
            gx = t1 * np.float32(HW)
            gy = t2 * np.float32(HW)
            gw = t3 * np.float32(HW)
            gh = t4 * np.float32(HW)
            gi = gx.astype(jnp.int32)
            gj = gy.astype(jnp.int32)
            tx = gx - gi.astype(jnp.float32)
            ty = gy - gj.astype(jnp.float32)
            ious = []
            for k in range(3):
                inter = jnp.maximum(
                    jnp.minimum(gw, _AW[k]) * jnp.minimum(gh, _AH[k]),
                    np.float32(0.0))
                union = gw * gh + _AA[k] - inter
                ious.append(inter / (union + _EPS16))
            best01 = jnp.where(ious[1] > ious[0], one_i, zero_i)
            m01 = jnp.where(ious[1] > ious[0], ious[1], ious[0])
            best = jnp.where(ious[2] > m01, 2 * one_i, best01)
            awb = jnp.where(best == 0, _AW[0], jnp.where(best == 1, _AW[1], _AW[2]))
            ahb = jnp.where(best == 0, _AH[0], jnp.where(best == 1, _AH[1], _AH[2]))
            rw = gw / awb + _EPS16
            rh = gh / ahb + _EPS16
            ssp = gj * HW + gi
            K = best * S + ssp
            cls = t0.astype(jnp.int32)
            n0 = jnp.where(ious[0] > np.float32(0.5), one_i, zero_i)
            n1 = jnp.where(ious[1] > np.float32(0.5), one_i, zero_i)
            n2 = jnp.where(ious[2] > np.float32(0.5), one_i, zero_i)
            nsum = n0 + n1 + n2
            z1 = jnp.where(nsum > 0, one_i, zero_i)
            z0 = jnp.where(nsum < 3, one_i, zero_i)
            met_v[pl.ds(64 + ch * 16, 16)] = tx
            met_v[pl.ds(128 + ch * 16, 16)] = ty
            met_v[pl.ds(192 + ch * 16, 16)] = rw
            met_v[pl.ds(256 + ch * 16, 16)] = rh
            Kc.append(K)
            Vc.append(valid)
            Z0c.append(z0)
            Z1c.append(z1)
            Sc.append(ssp)
            Cc.append(cls)
            ABc.append(bR + best * (BB * S) + ssp)
            PAc.append(bR + (best * BB + 5 + cls) * S + ssp)
        iotas = [lax.iota(jnp.int32, 16) + ch * 16 for ch in range(4)]

        PKr = [Kc[ch] * 128 + Cc[ch] for ch in range(4)]

        def make_killer(sch):
            def killer(tq, carry):
                kk = list(carry[0:4])
                kp = list(carry[4:8])
                k0 = list(carry[8:12])
                k1 = list(carry[12:16])
                tqv = jnp.full((16,), tq, jnp.int32)
                lane = tqv - sch * 16

                def pick(reg):
                    return reg.at[lane].get(mode="promise_in_bounds")

                kq = pick(Kc[sch])
                pq = pick(PKr[sch])
                sq = pick(Sc[sch])
                vq = pick(Vc[sch])
                zq0 = pick(Z0c[sch]) * vq
                zq1 = pick(Z1c[sch]) * vq
                for ch in range(4):
                    earlier = jnp.where(iotas[ch] < tqv, one_i, zero_i)
                    eqk = jnp.where(Kc[ch] == kq, one_i, zero_i) * earlier
                    eqp = jnp.where(PKr[ch] == pq, one_i, zero_i) * earlier
                    eqs = jnp.where(Sc[ch] == sq, one_i, zero_i) * earlier
                    kk[ch] = kk[ch] | (eqk * vq)
                    kp[ch] = kp[ch] | (eqp * vq)
                    k0[ch] = k0[ch] | (eqs * zq0 * Z0c[ch])
                    k1[ch] = k1[ch] | (eqs * zq1 * Z1c[ch])
                return tuple(kk + kp + k0 + k1)
            return killer

        carry = tuple([zero_i] * 16)
        for sch in range(4):
            lo = max(sch * 16, 1)
            hi = min(sch * 16 + 16, T)
            carry = lax.fori_loop(lo, hi, make_killer(sch), carry)

        for ch in range(4):
            sl = pl.ds(ch * 16, 16)
            win = Vc[ch] * (1 - carry[ch])
            pair = Vc[ch] * (1 - carry[4 + ch])
            nz0 = Vc[ch] * Z0c[ch] * (1 - carry[8 + ch])
            nz1 = Vc[ch] * Z1c[ch] * (1 - carry[12 + ch])
            met_v[sl] = win.astype(jnp.float32)
            met_v[pl.ds(320 + ch * 16, 16)] = pair.astype(jnp.float32)
            met_v[pl.ds(384 + ch * 16, 16)] = nz0.astype(jnp.float32)
            met_v[pl.ds(448 + ch * 16, 16)] = nz1.astype(jnp.float32)

        for c in range(BB):
            for ch in range(4):
                p = c * 64 + ch * 16
                idx_v[p // 128, pl.ds(p % 128, 16)] = ABc[ch] + c * S
        for ch in range(4):
            idx_v[42, pl.ds(64 + ch * 16, 16)] = PAc[ch]
            idx_v[43, pl.ds(ch * 16, 16)] = bR + 4 * S + Sc[ch]
            idx_v[43, pl.ds(64 + ch * 16, 16)] = bR + (BB + 4) * S + Sc[ch]

        cps = [pltpu.async_copy(flat_in.at[idx_v.at[r]], out_v.at[r], sem)
               for r in range(NROW)]
        for cp in cps:
            cp.wait()
        pltpu.sync_copy(out_v, gath_out.at[b])
        pltpu.sync_copy(met_v, meta_out.at[b])


_sc_assign = pl.kernel(
    _sc_assign_body,
    out_type=(
        jax.ShapeDtypeStruct((BS, NROW, 128), jnp.float32),
        jax.ShapeDtypeStruct((BS, 512), jnp.float32),
    ),
    mesh=plsc.VectorSubcoreMesh(
        core_axis_name="c", subcore_axis_name="s", num_cores=2, num_subcores=16),
    scratch_types=[
        pltpu.VMEM((5, TP), jnp.float32),
        pltpu.VMEM((NROW, 128), jnp.int32),
        pltpu.VMEM((NROW, 128), jnp.float32),
        pltpu.VMEM((512,), jnp.float32),
        pltpu.SemaphoreType.DMA,
    ],
)


def _sigm(x):
    return jnp.where(x >= 0,
                     1.0 / (1.0 + jnp.exp(-x)),
                     jnp.exp(x) / (1.0 + jnp.exp(x)))


def _tc_loss_body(inp_ref, gath_ref, meta_ref, out_ref, conf_v, sem):
    cps = []
    for i in range(BS * NA):
        off = ((i // NA) * NA * BB + (i % NA) * BB + 4) * S
        cps.append(pltpu.make_async_copy(
            inp_ref.at[pl.ds(off - off % 128, CPAD)], conf_v.at[i], sem))
    for cp in cps:
        cp.start()
    for cp in cps:
        cp.wait()
    one = np.float32(1.0)
    lane = lax.broadcasted_iota(jnp.int32, (BS * NA, CPAD), 1)
    row = lax.broadcasted_iota(jnp.int32, (BS * NA, CPAD), 0)
    bq = row // NA
    aq = row - NA * bq
    shift = 16 * ((bq * (NA * BB) + aq * BB + 4) & 7)
    inwin = (lane >= shift) & (lane < shift + S)
    confl = conf_v[...]
    pall = jnp.clip(_sigm(confl), _EPS12, one)
    dense = -jnp.sum(jnp.where(inwin, jnp.log(one - pall), np.float32(0.0)))

    g = gath_ref[...]
    m = meta_ref[...]
    win = m[:, 0]
    tx = m[:, 1]
    ty = m[:, 2]
    rw = jnp.where(win > 0, m[:, 3], one)
    rh = jnp.where(win > 0, m[:, 4], one)
    pairf = m[:, 5]
    nz0f = m[:, 6]
    nz1f = m[:, 7]

    def safe(v, flag):
        return jnp.where(flag > 0, v, np.float32(0.0))

    xl = safe(g[:, 0], win)
    yl = safe(g[:, 1], win)
    wl = g[:, 2]
    hl = g[:, 3]
    cfl = safe(g[:, 4], win)
    clsl = jnp.where(win[:, None, :] > 0, g[:, 5:85], np.float32(0.0))
    pairv = safe(g[:, 85], pairf)
    nz0v = safe(g[:, 86], nz0f)
    nz1v = safe(g[:, 87], nz1f)

    px = jnp.clip(_sigm(xl), _EPS12, one)
    py = jnp.clip(_sigm(yl), _EPS12, one)
    sx = jnp.sum(win * -(tx * jnp.log(px) + (one - tx) * jnp.log(one - px)))
    sy = jnp.sum(win * -(ty * jnp.log(py) + (one - ty) * jnp.log(one - py)))
    sw = jnp.sum(win * (wl - jnp.log(rw)) ** 2)
    sh = jnp.sum(win * (hl - jnp.log(rh)) ** 2)
    pcf = jnp.clip(_sigm(cfl), _EPS12, one)
    sconf = jnp.sum(win * -jnp.log(pcf))
    pcls = jnp.clip(_sigm(clsl), _EPS12, one)
    scls = -jnp.sum(win[:, None, :] * jnp.log(one - pcls))
    ppr = jnp.clip(_sigm(pairv), _EPS12, one)
    scls = scls + jnp.sum(pairf * (-jnp.log(ppr) + jnp.log(one - ppr)))
    p0 = jnp.clip(_sigm(nz0v), _EPS12, one)
    p1 = jnp.clip(_sigm(nz1v), _EPS12, one)
    nzcorr = jnp.sum(nz0f * -jnp.log(one - p0)) + jnp.sum(nz1f * -jnp.log(one - p1))
    nm = jnp.sum(win)

    n1 = np.float32(N1)
    loss_x = sx / n1
    loss_y = sy / n1
    loss_w = sw / n1
    loss_h = sh / n1
    loss_conf = sconf / n1
    loss_nconf = np.float32(0.5) * (dense - nzcorr) / n1
    loss_cls = scls / jnp.maximum(nm * NCLS, one)
    loss = (np.float32(2.5) * (loss_x + loss_y + loss_w + loss_h)
            + np.float32(10.0) * loss_conf + np.float32(3.0) * loss_nconf
            + np.float32(20.0) * loss_cls)

    ii = lax.broadcasted_iota(jnp.int32, (1, 128), 1)
    vals = (loss, loss_x, loss_y, loss_w, loss_h, loss_conf, loss_cls)
    v = jnp.zeros((1, 128), jnp.float32)
    for i, s in enumerate(vals):
        v = v + jnp.where(ii == i, s, np.float32(0.0))
    out_ref[...] = v


def kernel(input, targets):
    flat = input.reshape(BS * R)
    tgt = jnp.transpose(targets, (0, 2, 1))
    tgt = jnp.pad(tgt, ((0, 0), (0, 0), (0, TP - T)))
    gath, meta = _sc_assign(flat, tgt)
    gath = gath.reshape(BS, 2 * NROW, 64)
    meta = meta.reshape(BS, 8, 64)
    out = pl.pallas_call(
        _tc_loss_body,
        grid=(),
        in_specs=[
            pl.BlockSpec(memory_space=pl.ANY),
            pl.BlockSpec((BS, 2 * NROW, 64), lambda: (0, 0, 0)),
            pl.BlockSpec((BS, 8, 64), lambda: (0, 0, 0)),
        ],
        out_specs=pl.BlockSpec((1, 128), lambda: (0, 0)),
        out_shape=jax.ShapeDtypeStruct((1, 128), jnp.float32),
        scratch_shapes=[
            pltpu.VMEM((BS * NA, CPAD), jnp.float32),
            pltpu.SemaphoreType.DMA,
        ],
    )(flat, gath, meta)
    o = out[0]
    return (o[0], o[1], o[2], o[3], o[4], o[5], o[6])


# speedup vs baseline: 7.5634x; 1.5089x over previous
import functools

import jax
import jax.numpy as jnp
import numpy as np
from jax import lax
from jax.experimental import pallas as pl
from jax.experimental.pallas import tpu as pltpu
from jax.experimental.pallas import tpu_sc as plsc

BS = 16
NA = 3
NCLS = 80
BB = 5 + NCLS
HW = 52
S = HW * HW
R = NA * BB * S
N1 = float(BS * NA * S)
T = 50
TP = 64
NROW = 44
CPAD = S + 112

_AW = (np.float32(1.25), np.float32(2.0), np.float32(4.125))
_AH = (np.float32(1.625), np.float32(3.75), np.float32(2.875))
_AA = tuple(np.float32(a * b) for a, b in zip(_AW, _AH))
_EPS16 = np.float32(1e-16)
_EPS12 = np.float32(1e-12)


def _sc_assign_body(flat_in, tgt_hbm, gath_out, meta_out,
                    tgt_v, idx_v, out_v, met_v, sem):
    cidx = lax.axis_index("c")
    b = lax.axis_index("s")

    @pl.when(cidx == 0)
    def _():
        pltpu.sync_copy(tgt_hbm.at[b], tgt_v)
        bR = b * R

        one_i = jnp.ones((16,), jnp.int32)
        zero_i = jnp.zeros((16,), jnp.int32)
        Kc, Vc, Z0c, Z1c, Sc, Cc, ABc, PAc = [], [], [], [], [], [], [], []
        for ch in range(4):
            sl = pl.ds(ch * 16, 16)
            t0 = tgt_v[0, sl]
            t1 = tgt_v[1, sl]
            t2 = tgt_v[2, sl]
            t3 = tgt_v[3, sl]
            t4 = tgt_v[4, sl]
            valid = jnp.where((t0 + t1 + t2 + t3 + t4) != 0.0, one_i, zero_i)
            gx = t1 * np.float32(HW)
            gy = t2 * np.float32(HW)
            gw = t3 * np.float32(HW)
            gh = t4 * np.float32(HW)
            gi = gx.astype(jnp.int32)
            gj = gy.astype(jnp.int32)
            tx = gx - gi.astype(jnp.float32)
            ty = gy - gj.astype(jnp.float32)
            ious = []
            for k in range(3):
                inter = jnp.maximum(
                    jnp.minimum(gw, _AW[k]) * jnp.minimum(gh, _AH[k]),
                    np.float32(0.0))
                union = gw * gh + _AA[k] - inter
                ious.append(inter / (union + _EPS16))
            best01 = jnp.where(ious[1] > ious[0], one_i, zero_i)
            m01 = jnp.where(ious[1] > ious[0], ious[1], ious[0])
            best = jnp.where(ious[2] > m01, 2 * one_i, best01)
            awb = jnp.where(best == 0, _AW[0], jnp.where(best == 1, _AW[1], _AW[2]))
            ahb = jnp.where(best == 0, _AH[0], jnp.where(best == 1, _AH[1], _AH[2]))
            rw = gw / awb + _EPS16
            rh = gh / ahb + _EPS16
            ssp = gj * HW + gi
            K = best * S + ssp
            cls = t0.astype(jnp.int32)
            n0 = jnp.where(ious[0] > np.float32(0.5), one_i, zero_i)
            n1 = jnp.where(ious[1] > np.float32(0.5), one_i, zero_i)
            n2 = jnp.where(ious[2] > np.float32(0.5), one_i, zero_i)
            nsum = n0 + n1 + n2
            z1 = jnp.where(nsum > 0, one_i, zero_i)
            z0 = jnp.where(nsum < 3, one_i, zero_i)
            met_v[pl.ds(64 + ch * 16, 16)] = tx
            met_v[pl.ds(128 + ch * 16, 16)] = ty
            met_v[pl.ds(192 + ch * 16, 16)] = rw
            met_v[pl.ds(256 + ch * 16, 16)] = rh
            Kc.append(K)
            Vc.append(valid)
            Z0c.append(z0)
            Z1c.append(z1)
            Sc.append(ssp)
            Cc.append(cls)
            ABc.append(bR + best * (BB * S) + ssp)
            PAc.append(bR + (best * BB + 5 + cls) * S + ssp)
        iotas = [lax.iota(jnp.int32, 16) + ch * 16 for ch in range(4)]

        PKr = [Kc[ch] * 128 + Cc[ch] for ch in range(4)]

        def make_killer(sch):
            def killer(tq, carry):
                kk = list(carry[0:4])
                kp = list(carry[4:8])
                k0 = list(carry[8:12])
                k1 = list(carry[12:16])
                tqv = jnp.full((16,), tq, jnp.int32)
                lane = tqv - sch * 16

                def pick(reg):
                    return reg.at[lane].get(mode="promise_in_bounds")

                kq = pick(Kc[sch])
                pq = pick(PKr[sch])
                sq = pick(Sc[sch])
                vq = pick(Vc[sch])
                zq0 = pick(Z0c[sch]) * vq
                zq1 = pick(Z1c[sch]) * vq
                for ch in range(4):
                    earlier = jnp.where(iotas[ch] < tqv, one_i, zero_i)
                    eqk = jnp.where(Kc[ch] == kq, one_i, zero_i) * earlier
                    eqp = jnp.where(PKr[ch] == pq, one_i, zero_i) * earlier
                    eqs = jnp.where(Sc[ch] == sq, one_i, zero_i) * earlier
                    kk[ch] = kk[ch] | (eqk * vq)
                    kp[ch] = kp[ch] | (eqp * vq)
                    k0[ch] = k0[ch] | (eqs * zq0 * Z0c[ch])
                    k1[ch] = k1[ch] | (eqs * zq1 * Z1c[ch])
                return tuple(kk + kp + k0 + k1)
            return killer

        carry = tuple([zero_i] * 16)
        for sch in range(4):
            lo = max(sch * 16, 1)
            hi = min(sch * 16 + 16, T)
            carry = lax.fori_loop(lo, hi, make_killer(sch), carry)

        for ch in range(4):
            sl = pl.ds(ch * 16, 16)
            win = Vc[ch] * (1 - carry[ch])
            pair = Vc[ch] * (1 - carry[4 + ch])
            nz0 = Vc[ch] * Z0c[ch] * (1 - carry[8 + ch])
            nz1 = Vc[ch] * Z1c[ch] * (1 - carry[12 + ch])
            met_v[sl] = win.astype(jnp.float32)
            met_v[pl.ds(320 + ch * 16, 16)] = pair.astype(jnp.float32)
            met_v[pl.ds(384 + ch * 16, 16)] = nz0.astype(jnp.float32)
            met_v[pl.ds(448 + ch * 16, 16)] = nz1.astype(jnp.float32)

        for c in range(BB):
            for ch in range(4):
                p = c * 64 + ch * 16
                idx_v[p // 128, pl.ds(p % 128, 16)] = ABc[ch] + c * S
        for ch in range(4):
            idx_v[42, pl.ds(64 + ch * 16, 16)] = PAc[ch]
            idx_v[43, pl.ds(ch * 16, 16)] = bR + 4 * S + Sc[ch]
            idx_v[43, pl.ds(64 + ch * 16, 16)] = bR + (BB + 4) * S + Sc[ch]

        cps = [pltpu.async_copy(flat_in.at[idx_v.at[r]], out_v.at[r], sem)
               for r in range(NROW)]
        for cp in cps:
            cp.wait()
        pltpu.sync_copy(out_v, gath_out.at[b])
        pltpu.sync_copy(met_v, meta_out.at[b])


_sc_assign = pl.kernel(
    _sc_assign_body,
    out_type=(
        jax.ShapeDtypeStruct((BS, NROW, 128), jnp.float32),
        jax.ShapeDtypeStruct((BS, 512), jnp.float32),
    ),
    mesh=plsc.VectorSubcoreMesh(
        core_axis_name="c", subcore_axis_name="s", num_cores=2, num_subcores=16),
    scratch_types=[
        pltpu.VMEM((5, TP), jnp.float32),
        pltpu.VMEM((NROW, 128), jnp.int32),
        pltpu.VMEM((NROW, 128), jnp.float32),
        pltpu.VMEM((512,), jnp.float32),
        pltpu.SemaphoreType.DMA,
    ],
)


def _sigm(x):
    return jnp.where(x >= 0,
                     1.0 / (1.0 + jnp.exp(-x)),
                     jnp.exp(x) / (1.0 + jnp.exp(x)))


def _tc_dense_body(x_ref, out_ref):
    i = pl.program_id(0)
    one = np.float32(1.0)
    x = x_ref[0, 0]
    p = jnp.clip(_sigm(x), _EPS12, one)
    s = -jnp.sum(jnp.log(one - p))
    ii = lax.broadcasted_iota(jnp.int32, (1, 128), 1)
    sv = jnp.where(ii == 0, s, np.float32(0.0))

    @pl.when(i == 0)
    def _():
        out_ref[...] = jnp.zeros((1, 128), jnp.float32)

    out_ref[...] = out_ref[...] + sv


def _tc_loss_body(gath_ref, meta_ref, dense_ref, out_ref):
    one = np.float32(1.0)
    dense = dense_ref[0, 0]

    g = gath_ref[...]
    m = meta_ref[...]
    win = m[:, 0]
    tx = m[:, 1]
    ty = m[:, 2]
    rw = jnp.where(win > 0, m[:, 3], one)
    rh = jnp.where(win > 0, m[:, 4], one)
    pairf = m[:, 5]
    nz0f = m[:, 6]
    nz1f = m[:, 7]

    def safe(v, flag):
        return jnp.where(flag > 0, v, np.float32(0.0))

    xl = safe(g[:, 0], win)
    yl = safe(g[:, 1], win)
    wl = g[:, 2]
    hl = g[:, 3]
    cfl = safe(g[:, 4], win)
    clsl = jnp.where(win[:, None, :] > 0, g[:, 5:85], np.float32(0.0))
    pairv = safe(g[:, 85], pairf)
    nz0v = safe(g[:, 86], nz0f)
    nz1v = safe(g[:, 87], nz1f)

    px = jnp.clip(_sigm(xl), _EPS12, one)
    py = jnp.clip(_sigm(yl), _EPS12, one)
    sx = jnp.sum(win * -(tx * jnp.log(px) + (one - tx) * jnp.log(one - px)))
    sy = jnp.sum(win * -(ty * jnp.log(py) + (one - ty) * jnp.log(one - py)))
    sw = jnp.sum(win * (wl - jnp.log(rw)) ** 2)
    sh = jnp.sum(win * (hl - jnp.log(rh)) ** 2)
    pcf = jnp.clip(_sigm(cfl), _EPS12, one)
    sconf = jnp.sum(win * -jnp.log(pcf))
    pcls = jnp.clip(_sigm(clsl), _EPS12, one)
    scls = -jnp.sum(win[:, None, :] * jnp.log(one - pcls))
    ppr = jnp.clip(_sigm(pairv), _EPS12, one)
    scls = scls + jnp.sum(pairf * (-jnp.log(ppr) + jnp.log(one - ppr)))
    p0 = jnp.clip(_sigm(nz0v), _EPS12, one)
    p1 = jnp.clip(_sigm(nz1v), _EPS12, one)
    nzcorr = jnp.sum(nz0f * -jnp.log(one - p0)) + jnp.sum(nz1f * -jnp.log(one - p1))
    nm = jnp.sum(win)

    n1 = np.float32(N1)
    loss_x = sx / n1
    loss_y = sy / n1
    loss_w = sw / n1
    loss_h = sh / n1
    loss_conf = sconf / n1
    loss_nconf = np.float32(0.5) * (dense - nzcorr) / n1
    loss_cls = scls / jnp.maximum(nm * NCLS, one)
    loss = (np.float32(2.5) * (loss_x + loss_y + loss_w + loss_h)
            + np.float32(10.0) * loss_conf + np.float32(3.0) * loss_nconf
            + np.float32(20.0) * loss_cls)

    ii = lax.broadcasted_iota(jnp.int32, (1, 128), 1)
    vals = (loss, loss_x, loss_y, loss_w, loss_h, loss_conf, loss_cls)
    v = jnp.zeros((1, 128), jnp.float32)
    for i, s in enumerate(vals):
        v = v + jnp.where(ii == i, s, np.float32(0.0))
    out_ref[...] = v


def kernel(input, targets):
    flat = input.reshape(BS * R)
    tgt = jnp.transpose(targets, (0, 2, 1))
    tgt = jnp.pad(tgt, ((0, 0), (0, 0), (0, TP - T)))
    gath, meta = _sc_assign(flat, tgt)
    gath = gath.reshape(BS, 2 * NROW, 64)
    meta = meta.reshape(BS, 8, 64)
    dense = pl.pallas_call(
        _tc_dense_body,
        grid=(BS * NA,),
        in_specs=[
            pl.BlockSpec((1, 1, HW, HW),
                         lambda i: (i // NA, (i % NA) * BB + 4, 0, 0)),
        ],
        out_specs=pl.BlockSpec((1, 128), lambda i: (0, 0)),
        out_shape=jax.ShapeDtypeStruct((1, 128), jnp.float32),
    )(input)
    out = pl.pallas_call(
        _tc_loss_body,
        grid=(),
        in_specs=[
            pl.BlockSpec((BS, 2 * NROW, 64), lambda: (0, 0, 0)),
            pl.BlockSpec((BS, 8, 64), lambda: (0, 0, 0)),
            pl.BlockSpec((1, 128), lambda: (0, 0)),
        ],
        out_specs=pl.BlockSpec((1, 128), lambda: (0, 0)),
        out_shape=jax.ShapeDtypeStruct((1, 128), jnp.float32),
    )(gath, meta, dense)
    o = out[0]
    return (o[0], o[1], o[2], o[3], o[4], o[5], o[6])
